# Initial kernel scaffold; baseline (speedup 1.0000x reference)
import jax
import jax.numpy as jnp
from jax.experimental import pallas as pl


def kernel(position_ids, pos_enc):
    raise NotImplementedError("write your pallas kernel here")


# speedup vs baseline: 2.3065x; 2.3065x over previous
import functools

import jax
import jax.numpy as jnp
from jax import lax
from jax.experimental import pallas as pl
from jax.experimental.pallas import tpu as pltpu
from jax.experimental.pallas import tpu_sc as plsc

_NUM_CORES = 2
_NUM_SUBCORES = 16
_NW = _NUM_CORES * _NUM_SUBCORES
_CH = 32


@functools.partial(jax.jit, static_argnames=("n", "d"))
def _gather_rows(flat_ids, table, n, d):
    n_per_w = n // _NW
    nch = n_per_w // _CH
    mesh = plsc.VectorSubcoreMesh(core_axis_name="c", subcore_axis_name="s")

    @functools.partial(
        pl.kernel,
        mesh=mesh,
        out_type=jax.ShapeDtypeStruct((n, d), jnp.float32),
        scratch_types=[
            pltpu.VMEM((n_per_w,), jnp.int32),
            pltpu.VMEM((2, _CH, d), jnp.float32),
            pltpu.SemaphoreType.DMA,
            pltpu.SemaphoreType.DMA,
            pltpu.SemaphoreType.DMA,
            pltpu.SemaphoreType.DMA,
        ],
    )
    def k(ids_hbm, table_hbm, out_hbm, idx_v, rows_v, g0, g1, s0, s1):
        gsems = (g0, g1)
        ssems = (s0, s1)
        wid = lax.axis_index("s") * _NUM_CORES + lax.axis_index("c")
        base = pl.multiple_of(wid * n_per_w, n_per_w)
        pltpu.sync_copy(ids_hbm.at[pl.ds(base, n_per_w)], idx_v)

        def gather(g, b):
            off = pl.multiple_of(g * _CH, _CH)
            return pltpu.make_async_copy(
                table_hbm.at[idx_v.at[pl.ds(off, _CH)]], rows_v.at[b], gsems[b]
            )

        def scatter(g, b):
            off = pl.multiple_of(base + g * _CH, _CH)
            return pltpu.make_async_copy(
                rows_v.at[b], out_hbm.at[pl.ds(off, _CH)], ssems[b]
            )

        gather(0, 0).start()

        def step(i, _):
            for b in range(2):
                g = 2 * i + b
                gather(g, b).wait()
                scatter(g, b).start()

                @pl.when(g >= 1)
                def _():
                    scatter(g - 1, 1 - b).wait()

                @pl.when(g + 1 < nch)
                def _():
                    gather(g + 1, 1 - b).start()

            return 0

        lax.fori_loop(0, nch // 2, step, 0)
        scatter(nch - 1, 1).wait()

    return k(flat_ids, table)


def kernel(position_ids, pos_enc):
    b, s = position_ids.shape
    v, d = pos_enc.shape
    flat_ids = position_ids.reshape(b * s).astype(jnp.int32)
    out = _gather_rows(flat_ids, pos_enc, b * s, d)
    return out.reshape(b, s, d)


# speedup vs baseline: 2.3667x; 1.0261x over previous
import functools

import jax
import jax.numpy as jnp
from jax import lax
from jax.experimental import pallas as pl
from jax.experimental.pallas import tpu as pltpu
from jax.experimental.pallas import tpu_sc as plsc

_NUM_CORES = 2
_NUM_SUBCORES = 16
_NW = _NUM_CORES * _NUM_SUBCORES
_CH = 32


@functools.partial(jax.jit, static_argnames=("n", "d"))
def _gather_rows(flat_ids, table, n, d):
    n_per_w = n // _NW
    nch = n_per_w // _CH
    mesh = plsc.VectorSubcoreMesh(core_axis_name="c", subcore_axis_name="s")

    nbuf = 3

    @functools.partial(
        pl.kernel,
        mesh=mesh,
        out_type=jax.ShapeDtypeStruct((n, d), jnp.float32),
        scratch_types=[
            pltpu.VMEM((n_per_w,), jnp.int32),
            pltpu.VMEM((nbuf, _CH, d), jnp.float32),
            pltpu.SemaphoreType.DMA,
            pltpu.SemaphoreType.DMA,
            pltpu.SemaphoreType.DMA,
            pltpu.SemaphoreType.DMA,
            pltpu.SemaphoreType.DMA,
            pltpu.SemaphoreType.DMA,
        ],
    )
    def k(ids_hbm, table_hbm, out_hbm, idx_v, rows_v, g0, g1, g2, s0, s1, s2):
        gsems = (g0, g1, g2)
        ssems = (s0, s1, s2)
        wid = lax.axis_index("s") * _NUM_CORES + lax.axis_index("c")
        base = pl.multiple_of(wid * n_per_w, n_per_w)
        pltpu.sync_copy(ids_hbm.at[pl.ds(base, n_per_w)], idx_v)

        def gather(g, b):
            off = pl.multiple_of(g * _CH, _CH)
            return pltpu.make_async_copy(
                table_hbm.at[idx_v.at[pl.ds(off, _CH)]], rows_v.at[b], gsems[b]
            )

        def scatter(g, b):
            off = pl.multiple_of(base + g * _CH, _CH)
            return pltpu.make_async_copy(
                rows_v.at[b], out_hbm.at[pl.ds(off, _CH)], ssems[b]
            )

        gather(0, 0).start()
        gather(1, 1).start()

        n_main = (nch // nbuf) * nbuf

        def step(i, _):
            for b in range(nbuf):
                g = nbuf * i + b
                gather(g, b).wait()
                scatter(g, b).start()

                @pl.when(g >= 1)
                def _():
                    scatter(g - 1, (b + 2) % nbuf).wait()

                @pl.when(g + 2 < nch)
                def _():
                    gather(g + 2, (b + 2) % nbuf).start()

            return 0

        lax.fori_loop(0, n_main // nbuf, step, 0)
        for g in range(n_main, nch):
            b = g % nbuf
            gather(g, b).wait()
            scatter(g, b).start()
            scatter(g - 1, (g - 1) % nbuf).wait()
        scatter(nch - 1, (nch - 1) % nbuf).wait()

    return k(flat_ids, table)


def kernel(position_ids, pos_enc):
    b, s = position_ids.shape
    v, d = pos_enc.shape
    flat_ids = position_ids.reshape(b * s).astype(jnp.int32)
    out = _gather_rows(flat_ids, pos_enc, b * s, d)
    return out.reshape(b, s, d)
